# Initial kernel scaffold; baseline (speedup 1.0000x reference)
import functools

import jax
import jax.numpy as jnp
from jax import lax
from jax.experimental import pallas as pl
from jax.experimental.pallas import tpu as pltpu
from jax.experimental.pallas import tpu_sc as plsc

N_NODES = 10000
N_EDGES = 320000
D = 128

NC = 2
NS = 16
L = 16
NW = NC * NS

EDGES_PER_TILE = N_EDGES // NW
CHUNK = 80
CHUNKS_PER_TILE = EDGES_PER_TILE // CHUNK
ROWS_PER_TILE = N_NODES // NS


def _sc_segment_sum(x, src, dst, w, zeros_blk):
    mesh = plsc.VectorSubcoreMesh(core_axis_name="c", subcore_axis_name="s")

    @functools.partial(
        pl.kernel,
        out_type=jax.ShapeDtypeStruct((NC, N_NODES, D), jnp.float32),
        mesh=mesh,
        scratch_types=[
            pltpu.VMEM_SHARED((N_NODES, D), jnp.float32),
            pltpu.VMEM((EDGES_PER_TILE,), jnp.int32),
            pltpu.VMEM((CHUNKS_PER_TILE, CHUNK), jnp.int32),
            pltpu.VMEM((EDGES_PER_TILE,), jnp.float32),
            pltpu.VMEM((CHUNK, D), jnp.float32),
            pltpu.SemaphoreType.DMA,
        ],
    )
    def seg_kernel(x_hbm, src_hbm, dst_hbm, w_hbm, z_hbm, out_hbm,
                   acc, src_v, dst_v, w_v, rows_v, sem):
        cid = lax.axis_index("c")
        sid = lax.axis_index("s")
        wid = cid * NS + sid

        pltpu.sync_copy(z_hbm, acc.at[pl.ds(sid * ROWS_PER_TILE, ROWS_PER_TILE)])

        pltpu.sync_copy(src_hbm.at[wid], src_v)
        pltpu.sync_copy(dst_hbm.at[wid], dst_v)
        pltpu.sync_copy(w_hbm.at[wid], w_v)
        plsc.subcore_barrier()

        @pl.loop(0, CHUNKS_PER_TILE)
        def _chunk(c):
            base = c * CHUNK
            pltpu.async_copy(x_hbm.at[src_v.at[pl.ds(base, CHUNK)]], rows_v,
                             sem).wait()

            @pl.loop(0, CHUNK, unroll=4)
            def _edge(e):
                wb = plsc.load_gather(w_v, [jnp.full((L,), base + e, jnp.int32)])
                for k in range(D // L):
                    sl = pl.ds(k * L, L)
                    rows_v[e, sl] = rows_v[e, sl] * wb

            pltpu.sync_copy(rows_v, acc.at[dst_v.at[c]], add=True)

        plsc.subcore_barrier()
        pltpu.sync_copy(acc.at[pl.ds(sid * ROWS_PER_TILE, ROWS_PER_TILE)],
                        out_hbm.at[cid].at[pl.ds(sid * ROWS_PER_TILE,
                                                 ROWS_PER_TILE)])

    return seg_kernel(x, src, dst, w, zeros_blk)


_TC_BLK = 1000


def _combine_body(p_ref, x_ref, wrel_ref, b_ref, wroot_ref, o_ref):
    agg = p_ref[0] + p_ref[1]
    y = (jnp.dot(agg, wrel_ref[...], preferred_element_type=jnp.float32)
         + jnp.dot(x_ref[...], wroot_ref[...], preferred_element_type=jnp.float32)
         + b_ref[...])
    y = jnp.maximum(y, 0.0)
    o_ref[...] = jnp.maximum(y + x_ref[...], 0.0)


def _tc_combine(partials, x, W_rel, b_rel, W_root):
    grid = (N_NODES // _TC_BLK,)
    return pl.pallas_call(
        _combine_body,
        grid=grid,
        in_specs=[
            pl.BlockSpec((NC, _TC_BLK, D), lambda i: (0, i, 0)),
            pl.BlockSpec((_TC_BLK, D), lambda i: (i, 0)),
            pl.BlockSpec((D, D), lambda i: (0, 0)),
            pl.BlockSpec((1, D), lambda i: (0, 0)),
            pl.BlockSpec((D, D), lambda i: (0, 0)),
        ],
        out_specs=pl.BlockSpec((_TC_BLK, D), lambda i: (i, 0)),
        out_shape=jax.ShapeDtypeStruct((N_NODES, D), jnp.float32),
    )(partials, x, W_rel, b_rel.reshape(1, D), W_root)


def kernel(x, edge_index, edge_weight, batch, W_rel, b_rel, W_root):
    src = edge_index[0].astype(jnp.int32).reshape(NW, EDGES_PER_TILE)
    dst = edge_index[1].astype(jnp.int32).reshape(NW, CHUNKS_PER_TILE, CHUNK)
    w = edge_weight.astype(jnp.float32).reshape(NW, EDGES_PER_TILE)
    zeros_blk = jnp.zeros((ROWS_PER_TILE, D), jnp.float32)
    partials = _sc_segment_sum(x, src, dst, w, zeros_blk)
    return _tc_combine(partials, x, W_rel, b_rel, W_root)


# speedup vs baseline: 4.6343x; 4.6343x over previous
import dataclasses
import functools

import jax
import jax.numpy as jnp
from jax import lax
from jax.experimental import pallas as pl
from jax.experimental.pallas import tpu as pltpu
from jax.experimental.pallas import tpu_sc as plsc

N_NODES = 10000
N_EDGES = 320000
D = 128

NC = 2
NS = 16
L = 16
NW = NC * NS

EDGES_PER_TILE = N_EDGES // NW
CHUNK = 80
CHUNKS_PER_TILE = EDGES_PER_TILE // CHUNK
N_PAD = 10240
ROWS_PER_TILE = N_PAD // NS


def _sc_segment_sum(x, src, dst, w, zeros_blk):
    mesh = plsc.VectorSubcoreMesh(core_axis_name="c", subcore_axis_name="s")
    cp = pltpu.CompilerParams()
    if "needs_layout_passes" in pltpu.CompilerParams.__dataclass_fields__:
        cp = dataclasses.replace(cp, needs_layout_passes=False)

    @functools.partial(
        pl.kernel,
        compiler_params=cp,
        out_type=jax.ShapeDtypeStruct((NC, N_PAD, D), jnp.float32),
        mesh=mesh,
        scratch_types=[
            pltpu.VMEM_SHARED((N_PAD, D), jnp.float32),
            pltpu.VMEM((EDGES_PER_TILE,), jnp.int32),
            pltpu.VMEM((CHUNKS_PER_TILE, CHUNK), jnp.int32),
            pltpu.VMEM((EDGES_PER_TILE,), jnp.float32),
            pltpu.VMEM((CHUNK, D), jnp.float32),
            pltpu.SemaphoreType.DMA,
        ],
    )
    def seg_kernel(x_hbm, src_hbm, dst_hbm, w_hbm, z_hbm, out_hbm,
                   acc, src_v, dst_v, w_v, rows_v, sem):
        cid = lax.axis_index("c")
        sid = lax.axis_index("s")
        wid = cid * NS + sid

        pltpu.sync_copy(z_hbm, acc.at[pl.ds(sid * ROWS_PER_TILE, ROWS_PER_TILE)])

        pltpu.sync_copy(src_hbm.at[wid], src_v)
        pltpu.sync_copy(dst_hbm.at[wid], dst_v)
        pltpu.sync_copy(w_hbm.at[wid], w_v)
        plsc.subcore_barrier()

        @pl.loop(0, CHUNKS_PER_TILE)
        def _chunk(c):
            base = c * CHUNK
            pltpu.async_copy(x_hbm.at[src_v.at[pl.ds(base, CHUNK)]], rows_v,
                             sem).wait()

            @pl.loop(0, CHUNK, unroll=4)
            def _edge(e):
                wb = plsc.load_gather(w_v, [jnp.full((L,), base + e, jnp.int32)])
                for k in range(D // L):
                    sl = pl.ds(k * L, L)
                    rows_v[e, sl] = rows_v[e, sl] * wb

            pltpu.sync_copy(rows_v, acc.at[dst_v.at[c]], add=True)

        plsc.subcore_barrier()
        pltpu.sync_copy(acc.at[pl.ds(sid * ROWS_PER_TILE, ROWS_PER_TILE)],
                        out_hbm.at[cid].at[pl.ds(sid * ROWS_PER_TILE,
                                                 ROWS_PER_TILE)])

    return seg_kernel(x, src, dst, w, zeros_blk)


_TC_BLK = 80


def _combine_body(p_ref, x_ref, wrel_ref, b_ref, wroot_ref, o_ref):
    agg = p_ref[0] + p_ref[1]
    y = (jnp.dot(agg, wrel_ref[...], preferred_element_type=jnp.float32)
         + jnp.dot(x_ref[...], wroot_ref[...], preferred_element_type=jnp.float32)
         + b_ref[...])
    y = jnp.maximum(y, 0.0)
    o_ref[...] = jnp.maximum(y + x_ref[...], 0.0)


def _tc_combine(partials, x, W_rel, b_rel, W_root):
    grid = (N_NODES // _TC_BLK,)
    return pl.pallas_call(
        _combine_body,
        grid=grid,
        in_specs=[
            pl.BlockSpec((NC, _TC_BLK, D), lambda i: (0, i, 0)),
            pl.BlockSpec((_TC_BLK, D), lambda i: (i, 0)),
            pl.BlockSpec((D, D), lambda i: (0, 0)),
            pl.BlockSpec((1, D), lambda i: (0, 0)),
            pl.BlockSpec((D, D), lambda i: (0, 0)),
        ],
        out_specs=pl.BlockSpec((_TC_BLK, D), lambda i: (i, 0)),
        out_shape=jax.ShapeDtypeStruct((N_NODES, D), jnp.float32),
    )(partials, x, W_rel, b_rel.reshape(1, D), W_root)


def kernel(x, edge_index, edge_weight, batch, W_rel, b_rel, W_root):
    src = edge_index[0].astype(jnp.int32).reshape(NW, EDGES_PER_TILE)
    dst = edge_index[1].astype(jnp.int32).reshape(NW, CHUNKS_PER_TILE, CHUNK)
    w = edge_weight.astype(jnp.float32).reshape(NW, EDGES_PER_TILE)
    zeros_blk = jnp.zeros((ROWS_PER_TILE, D), jnp.float32)
    partials = _sc_segment_sum(x, src, dst, w, zeros_blk)
    return _tc_combine(partials, x, W_rel, b_rel, W_root)


# speedup vs baseline: 6.7110x; 1.4481x over previous
import dataclasses
import functools

import jax
import jax.numpy as jnp
from jax import lax
from jax.experimental import pallas as pl
from jax.experimental.pallas import tpu as pltpu
from jax.experimental.pallas import tpu_sc as plsc

N_NODES = 10000
N_EDGES = 320000
D = 128

NC = 2
NS = 16
L = 16
NW = NC * NS

EDGES_PER_TILE = N_EDGES // NW
CHUNK = 80
CHUNKS_PER_TILE = EDGES_PER_TILE // CHUNK
N_PAD = 10240
ROWS_PER_TILE = N_PAD // NS


def _sc_segment_sum(x, src, dw, zeros_blk):
    mesh = plsc.VectorSubcoreMesh(core_axis_name="c", subcore_axis_name="s")
    cp = pltpu.CompilerParams()
    if "needs_layout_passes" in pltpu.CompilerParams.__dataclass_fields__:
        cp = dataclasses.replace(cp, needs_layout_passes=False)

    @functools.partial(
        pl.kernel,
        compiler_params=cp,
        out_type=jax.ShapeDtypeStruct((NC, N_PAD, D), jnp.float32),
        mesh=mesh,
        scratch_types=[
            pltpu.VMEM_SHARED((N_PAD, D), jnp.float32),
            pltpu.VMEM((EDGES_PER_TILE,), jnp.int32),
            pltpu.VMEM((2, CHUNK), jnp.int32),
            pltpu.VMEM((2, CHUNK), jnp.int32),
            pltpu.VMEM((CHUNK, D), jnp.float32),
            pltpu.VMEM((CHUNK, D), jnp.float32),
            pltpu.SemaphoreType.DMA,
            pltpu.SemaphoreType.DMA,
            pltpu.SemaphoreType.DMA,
            pltpu.SemaphoreType.DMA,
            pltpu.SemaphoreType.DMA,
            pltpu.SemaphoreType.DMA,
        ],
    )
    def seg_kernel(x_hbm, src_hbm, dw_hbm, z_hbm, out_hbm,
                   acc, src_v, dw0, dw1, rows0, rows1, g0, g1, s0, s1, d0, d1):
        cid = lax.axis_index("c")
        sid = lax.axis_index("s")
        wid = cid * NS + sid

        pltpu.sync_copy(z_hbm, acc.at[pl.ds(sid * ROWS_PER_TILE, ROWS_PER_TILE)])

        pltpu.sync_copy(src_hbm.at[wid], src_v)
        plsc.subcore_barrier()

        my_dw = dw_hbm.at[wid]

        def issue_gather(c, rows, dw, gsem, dsem):
            pltpu.async_copy(my_dw.at[c], dw, dsem)
            pltpu.async_copy(x_hbm.at[src_v.at[pl.ds(c * CHUNK, CHUNK)]],
                             rows, gsem)

        def wait_gather(rows, dw, gsem, dsem):
            pltpu.make_async_copy(my_dw.at[0], dw, dsem).wait()
            pltpu.make_async_copy(x_hbm.at[src_v.at[pl.ds(0, CHUNK)]],
                                  rows, gsem).wait()

        def issue_scatter(rows, dw, sem):
            pltpu.async_copy(rows, acc.at[dw.at[0]], sem, add=True)

        def wait_scatter(rows, dw, sem):
            pltpu.make_async_copy(rows, acc.at[dw.at[0]], sem).wait()

        def scale(rows, dw):
            @plsc.parallel_loop(0, CHUNK, unroll=4)
            def _edge(e):
                one = jnp.full((L,), 1, jnp.int32)
                wb = plsc.bitcast(
                    plsc.load_gather(dw, [one, jnp.full((L,), e, jnp.int32)]),
                    jnp.float32)
                for k in range(D // L):
                    sl = pl.ds(k * L, L)
                    rows[e, sl] = rows[e, sl] * wb

        issue_gather(0, rows0, dw0, g0, d0)
        wait_gather(rows0, dw0, g0, d0)
        issue_gather(1, rows1, dw1, g1, d1)
        scale(rows0, dw0)
        issue_scatter(rows0, dw0, s0)

        @pl.loop(0, (CHUNKS_PER_TILE - 1) // 2)
        def _pair(p):
            c1 = 2 * p + 1
            wait_gather(rows1, dw1, g1, d1)
            wait_scatter(rows0, dw0, s0)
            issue_gather(c1 + 1, rows0, dw0, g0, d0)
            scale(rows1, dw1)
            issue_scatter(rows1, dw1, s1)

            c2 = 2 * p + 2
            wait_gather(rows0, dw0, g0, d0)
            wait_scatter(rows1, dw1, s1)

            @pl.when(c2 < CHUNKS_PER_TILE - 1)
            def _():
                issue_gather(c2 + 1, rows1, dw1, g1, d1)

            scale(rows0, dw0)
            issue_scatter(rows0, dw0, s0)

        wait_scatter(rows0, dw0, s0)
        plsc.subcore_barrier()
        pltpu.sync_copy(acc.at[pl.ds(sid * ROWS_PER_TILE, ROWS_PER_TILE)],
                        out_hbm.at[cid].at[pl.ds(sid * ROWS_PER_TILE,
                                                 ROWS_PER_TILE)])

    return seg_kernel(x, src, dw, zeros_blk)


_TC_BLK = 80


def _combine_body(p_ref, x_ref, wrel_ref, b_ref, wroot_ref, o_ref):
    agg = p_ref[0] + p_ref[1]
    y = (jnp.dot(agg, wrel_ref[...], preferred_element_type=jnp.float32)
         + jnp.dot(x_ref[...], wroot_ref[...], preferred_element_type=jnp.float32)
         + b_ref[...])
    y = jnp.maximum(y, 0.0)
    o_ref[...] = jnp.maximum(y + x_ref[...], 0.0)


def _tc_combine(partials, x, W_rel, b_rel, W_root):
    grid = (N_NODES // _TC_BLK,)
    return pl.pallas_call(
        _combine_body,
        grid=grid,
        in_specs=[
            pl.BlockSpec((NC, _TC_BLK, D), lambda i: (0, i, 0)),
            pl.BlockSpec((_TC_BLK, D), lambda i: (i, 0)),
            pl.BlockSpec((D, D), lambda i: (0, 0)),
            pl.BlockSpec((1, D), lambda i: (0, 0)),
            pl.BlockSpec((D, D), lambda i: (0, 0)),
        ],
        out_specs=pl.BlockSpec((_TC_BLK, D), lambda i: (i, 0)),
        out_shape=jax.ShapeDtypeStruct((N_NODES, D), jnp.float32),
    )(partials, x, W_rel, b_rel.reshape(1, D), W_root)


def kernel(x, edge_index, edge_weight, batch, W_rel, b_rel, W_root):
    src = edge_index[0].astype(jnp.int32).reshape(NW, EDGES_PER_TILE)
    dst = edge_index[1].astype(jnp.int32).reshape(NW, CHUNKS_PER_TILE, CHUNK)
    w_bits = lax.bitcast_convert_type(
        edge_weight.astype(jnp.float32), jnp.int32
    ).reshape(NW, CHUNKS_PER_TILE, CHUNK)
    dw = jnp.stack([dst, w_bits], axis=2)
    zeros_blk = jnp.zeros((ROWS_PER_TILE, D), jnp.float32)
    partials = _sc_segment_sum(x, src, dw, zeros_blk)
    return _tc_combine(partials, x, W_rel, b_rel, W_root)


# speedup vs baseline: 8.5626x; 1.2759x over previous
import dataclasses
import functools

import jax
import jax.numpy as jnp
from jax import lax
from jax.experimental import pallas as pl
from jax.experimental.pallas import tpu as pltpu
from jax.experimental.pallas import tpu_sc as plsc

N_NODES = 10000
N_EDGES = 320000
D = 128

NC = 2
NS = 16
L = 16
NW = NC * NS

EDGES_PER_TILE = N_EDGES // NW
CHUNK = 80
CHUNKS_PER_TILE = EDGES_PER_TILE // CHUNK
N_PAD = 10240
ROWS_PER_TILE = N_PAD // NS


def _sc_segment_sum(x, src, dw, zeros_blk):
    mesh = plsc.VectorSubcoreMesh(core_axis_name="c", subcore_axis_name="s")
    cp = pltpu.CompilerParams()
    if "needs_layout_passes" in pltpu.CompilerParams.__dataclass_fields__:
        cp = dataclasses.replace(cp, needs_layout_passes=False)

    @functools.partial(
        pl.kernel,
        compiler_params=cp,
        out_type=jax.ShapeDtypeStruct((NC, N_PAD, D), jnp.float32),
        mesh=mesh,
        scratch_types=[
            pltpu.VMEM_SHARED((N_PAD, D), jnp.float32),
            pltpu.VMEM((EDGES_PER_TILE,), jnp.int32),
            [pltpu.VMEM((2, CHUNK), jnp.int32)] * 6,
            [pltpu.VMEM((CHUNK, D), jnp.float32)] * 3,
            [pltpu.SemaphoreType.DMA] * 3,
            [pltpu.SemaphoreType.DMA] * 3,
            [pltpu.SemaphoreType.DMA] * 6,
        ],
    )
    def seg_kernel(x_hbm, src_hbm, dw_hbm, z_hbm, out_hbm,
                   acc, src_v, dwb, rows, g, s, d):
        cid = lax.axis_index("c")
        sid = lax.axis_index("s")
        wid = cid * NS + sid

        pltpu.sync_copy(z_hbm, acc.at[pl.ds(sid * ROWS_PER_TILE, ROWS_PER_TILE)])

        pltpu.sync_copy(src_hbm.at[wid], src_v)
        plsc.subcore_barrier()

        my_dw = dw_hbm.at[wid]

        def issue_dw(c, i):
            pltpu.async_copy(my_dw.at[c], dwb[i], d[i])

        def wait_dw(i):
            pltpu.make_async_copy(my_dw.at[0], dwb[i], d[i]).wait()

        def issue_gather(c, j):
            pltpu.async_copy(x_hbm.at[src_v.at[pl.ds(c * CHUNK, CHUNK)]],
                             rows[j], g[j])

        def wait_gather(j):
            pltpu.make_async_copy(x_hbm.at[src_v.at[pl.ds(0, CHUNK)]],
                                  rows[j], g[j]).wait()

        def issue_scatter(j, i):
            pltpu.async_copy(rows[j], acc.at[dwb[i].at[0]], s[j], add=True)

        def wait_scatter(j):
            pltpu.make_async_copy(rows[j], acc.at[dwb[0].at[0]], s[j]).wait()

        def scale(j, i):
            @plsc.parallel_loop(0, CHUNK, unroll=4)
            def _edge(e):
                one = jnp.full((L,), 1, jnp.int32)
                wb = plsc.bitcast(
                    plsc.load_gather(dwb[i],
                                     [one, jnp.full((L,), e, jnp.int32)]),
                    jnp.float32)
                for k in range(D // L):
                    sl = pl.ds(k * L, L)
                    rows[j][e, sl] = rows[j][e, sl] * wb

        M = CHUNKS_PER_TILE

        def body(c, rj, di, *, do_wait_scatter=True, issue_g=True,
                 issue_d=True, gc=None, dc=None):
            wait_gather(rj)
            if do_wait_scatter:
                wait_scatter((rj + 1) % 3)
            if issue_g:
                issue_gather(gc, (rj + 1) % 3)
            wait_dw(di)
            if issue_d:
                issue_dw(dc, (di + 2) % 6)
            scale(rj, di)
            issue_scatter(rj, di)

        issue_dw(0, 0)
        issue_dw(1, 1)
        issue_gather(0, 0)
        body(0, 0, 0, do_wait_scatter=False, gc=1, dc=2)
        body(1, 1, 1, do_wait_scatter=False, gc=2, dc=3)

        @pl.loop(0, (M - 5) // 6)
        def _steady(p):
            c0 = 6 * p + 2
            for q in range(6):
                rj = (2 + q) % 3
                di = (2 + q) % 6
                body(c0 + q, rj, di, gc=c0 + q + 1, dc=c0 + q + 2)

        body(M - 3, (M - 3) % 3, (M - 3) % 6, gc=M - 2, dc=M - 1)
        body(M - 2, (M - 2) % 3, (M - 2) % 6, gc=M - 1, issue_d=False)
        body(M - 1, (M - 1) % 3, (M - 1) % 6, issue_g=False, issue_d=False)

        wait_scatter((M - 2) % 3)
        wait_scatter((M - 1) % 3)
        plsc.subcore_barrier()
        pltpu.sync_copy(acc.at[pl.ds(sid * ROWS_PER_TILE, ROWS_PER_TILE)],
                        out_hbm.at[cid].at[pl.ds(sid * ROWS_PER_TILE,
                                                 ROWS_PER_TILE)])

    return seg_kernel(x, src, dw, zeros_blk)


_TC_BLK = 2000


def _combine_body(p_ref, x_ref, wrel_ref, b_ref, wroot_ref, o_ref):
    agg = p_ref[0] + p_ref[1]
    y = (jnp.dot(agg, wrel_ref[...], preferred_element_type=jnp.float32)
         + jnp.dot(x_ref[...], wroot_ref[...], preferred_element_type=jnp.float32)
         + b_ref[...])
    y = jnp.maximum(y, 0.0)
    o_ref[...] = jnp.maximum(y + x_ref[...], 0.0)


def _tc_combine(partials, x, W_rel, b_rel, W_root):
    grid = (N_NODES // _TC_BLK,)
    return pl.pallas_call(
        _combine_body,
        grid=grid,
        in_specs=[
            pl.BlockSpec((NC, _TC_BLK, D), lambda i: (0, i, 0)),
            pl.BlockSpec((_TC_BLK, D), lambda i: (i, 0)),
            pl.BlockSpec((D, D), lambda i: (0, 0)),
            pl.BlockSpec((1, D), lambda i: (0, 0)),
            pl.BlockSpec((D, D), lambda i: (0, 0)),
        ],
        out_specs=pl.BlockSpec((_TC_BLK, D), lambda i: (i, 0)),
        out_shape=jax.ShapeDtypeStruct((N_NODES, D), jnp.float32),
    )(partials, x, W_rel, b_rel.reshape(1, D), W_root)


def kernel(x, edge_index, edge_weight, batch, W_rel, b_rel, W_root):
    src = edge_index[0].astype(jnp.int32).reshape(NW, EDGES_PER_TILE)
    dst = edge_index[1].astype(jnp.int32).reshape(NW, CHUNKS_PER_TILE, CHUNK)
    w_bits = lax.bitcast_convert_type(
        edge_weight.astype(jnp.float32), jnp.int32
    ).reshape(NW, CHUNKS_PER_TILE, CHUNK)
    dw = jnp.stack([dst, w_bits], axis=2)
    zeros_blk = jnp.zeros((ROWS_PER_TILE, D), jnp.float32)
    partials = _sc_segment_sum(x, src, dw, zeros_blk)
    return _tc_combine(partials, x, W_rel, b_rel, W_root)


# speedup vs baseline: 9.0194x; 1.0534x over previous
import dataclasses
import functools

import jax
import jax.numpy as jnp
from jax import lax
from jax.experimental import pallas as pl
from jax.experimental.pallas import tpu as pltpu
from jax.experimental.pallas import tpu_sc as plsc

N_NODES = 10000
N_EDGES = 320000
D = 128

NC = 2
NS = 16
L = 16
NW = NC * NS

EDGES_PER_TILE = N_EDGES // NW
CHUNK = 80
CHUNKS_PER_TILE = EDGES_PER_TILE // CHUNK
N_PAD = 10240
ROWS_PER_TILE = N_PAD // NS


def _sc_segment_sum(x, src, dst, w):
    mesh = plsc.VectorSubcoreMesh(core_axis_name="c", subcore_axis_name="s")
    cp = pltpu.CompilerParams()
    if "needs_layout_passes" in pltpu.CompilerParams.__dataclass_fields__:
        cp = dataclasses.replace(cp, needs_layout_passes=False)

    @functools.partial(
        pl.kernel,
        compiler_params=cp,
        out_type=jax.ShapeDtypeStruct((NC, N_PAD, D), jnp.float32),
        mesh=mesh,
        scratch_types=[
            pltpu.VMEM_SHARED((N_PAD, D), jnp.float32),
            pltpu.VMEM((EDGES_PER_TILE,), jnp.int32),
            [pltpu.VMEM((1, CHUNK), jnp.int32)] * 6,
            [pltpu.VMEM((1, CHUNK), jnp.float32)] * 6,
            [pltpu.VMEM((CHUNK, D), jnp.float32)] * 3,
            [pltpu.SemaphoreType.DMA] * 3,
            [pltpu.SemaphoreType.DMA] * 3,
            [pltpu.SemaphoreType.DMA] * 6,
        ],
    )
    def seg_kernel(x_hbm, src_hbm, dst_hbm, w_hbm, out_hbm,
                   acc, src_v, dstb, wgtb, rows, g, s, d):
        cid = lax.axis_index("c")
        sid = lax.axis_index("s")
        wid = cid * NS + sid

        zv = jnp.zeros((L,), jnp.float32)

        @pl.loop(0, CHUNK)
        def _zrow(r):
            for k in range(D // L):
                rows[0][r, pl.ds(k * L, L)] = zv

        for q in range(ROWS_PER_TILE // CHUNK):
            pltpu.sync_copy(
                rows[0],
                acc.at[pl.ds(sid * ROWS_PER_TILE + q * CHUNK, CHUNK)])

        pltpu.sync_copy(src_hbm.at[wid], src_v)
        plsc.subcore_barrier()

        my_dst = dst_hbm.at[wid]
        my_w = w_hbm.at[wid]

        def issue_dw(c, i):
            pltpu.async_copy(my_dst.at[c], dstb[i], d[i])
            pltpu.async_copy(my_w.at[c], wgtb[i], d[i])

        def wait_dw(i):
            pltpu.make_async_copy(my_dst.at[0], dstb[i], d[i]).wait()
            pltpu.make_async_copy(my_w.at[0], wgtb[i], d[i]).wait()

        def issue_gather(c, j):
            pltpu.async_copy(x_hbm.at[src_v.at[pl.ds(c * CHUNK, CHUNK)]],
                             rows[j], g[j])

        def wait_gather(j):
            pltpu.make_async_copy(x_hbm.at[src_v.at[pl.ds(0, CHUNK)]],
                                  rows[j], g[j]).wait()

        def issue_scatter(j, i):
            pltpu.async_copy(rows[j], acc.at[dstb[i].at[0]], s[j], add=True)

        def wait_scatter(j):
            pltpu.make_async_copy(rows[j], acc.at[dstb[0].at[0]], s[j]).wait()

        def scale(j, i):
            @plsc.parallel_loop(0, CHUNK, unroll=4)
            def _edge(e):
                wb = plsc.load_gather(
                    wgtb[i],
                    [jnp.zeros((L,), jnp.int32), jnp.full((L,), e, jnp.int32)])
                for k in range(D // L):
                    sl = pl.ds(k * L, L)
                    rows[j][e, sl] = rows[j][e, sl] * wb

        M = CHUNKS_PER_TILE

        def body(c, rj, di, *, do_wait_scatter=True, issue_g=True,
                 issue_d=True, gc=None, dc=None):
            wait_gather(rj)
            if do_wait_scatter:
                wait_scatter((rj + 1) % 3)
            if issue_g:
                issue_gather(gc, (rj + 1) % 3)
            wait_dw(di)
            if issue_d:
                issue_dw(dc, (di + 2) % 6)
            scale(rj, di)
            issue_scatter(rj, di)

        issue_dw(0, 0)
        issue_dw(1, 1)
        issue_gather(0, 0)
        body(0, 0, 0, do_wait_scatter=False, gc=1, dc=2)
        body(1, 1, 1, do_wait_scatter=False, gc=2, dc=3)

        @pl.loop(0, (M - 5) // 6)
        def _steady(p):
            c0 = 6 * p + 2
            for q in range(6):
                rj = (2 + q) % 3
                di = (2 + q) % 6
                body(c0 + q, rj, di, gc=c0 + q + 1, dc=c0 + q + 2)

        body(M - 3, (M - 3) % 3, (M - 3) % 6, gc=M - 2, dc=M - 1)
        body(M - 2, (M - 2) % 3, (M - 2) % 6, gc=M - 1, issue_d=False)
        body(M - 1, (M - 1) % 3, (M - 1) % 6, issue_g=False, issue_d=False)

        wait_scatter((M - 2) % 3)
        wait_scatter((M - 1) % 3)
        plsc.subcore_barrier()
        pltpu.sync_copy(acc.at[pl.ds(sid * ROWS_PER_TILE, ROWS_PER_TILE)],
                        out_hbm.at[cid].at[pl.ds(sid * ROWS_PER_TILE,
                                                 ROWS_PER_TILE)])

    return seg_kernel(x, src, dst, w)


_TC_BLK = 2000


def _combine_body(p_ref, x_ref, wrel_ref, b_ref, wroot_ref, o_ref):
    agg = p_ref[0] + p_ref[1]
    y = (jnp.dot(agg, wrel_ref[...], preferred_element_type=jnp.float32)
         + jnp.dot(x_ref[...], wroot_ref[...], preferred_element_type=jnp.float32)
         + b_ref[...])
    y = jnp.maximum(y, 0.0)
    o_ref[...] = jnp.maximum(y + x_ref[...], 0.0)


def _tc_combine(partials, x, W_rel, b_rel, W_root):
    grid = (N_NODES // _TC_BLK,)
    return pl.pallas_call(
        _combine_body,
        grid=grid,
        in_specs=[
            pl.BlockSpec((NC, _TC_BLK, D), lambda i: (0, i, 0)),
            pl.BlockSpec((_TC_BLK, D), lambda i: (i, 0)),
            pl.BlockSpec((D, D), lambda i: (0, 0)),
            pl.BlockSpec((1, D), lambda i: (0, 0)),
            pl.BlockSpec((D, D), lambda i: (0, 0)),
        ],
        out_specs=pl.BlockSpec((_TC_BLK, D), lambda i: (i, 0)),
        out_shape=jax.ShapeDtypeStruct((N_NODES, D), jnp.float32),
    )(partials, x, W_rel, b_rel.reshape(1, D), W_root)


def kernel(x, edge_index, edge_weight, batch, W_rel, b_rel, W_root):
    src = edge_index[0].astype(jnp.int32).reshape(NW, EDGES_PER_TILE)
    dst = edge_index[1].astype(jnp.int32).reshape(
        NW, CHUNKS_PER_TILE, 1, CHUNK)
    w = edge_weight.astype(jnp.float32).reshape(NW, CHUNKS_PER_TILE, 1, CHUNK)
    partials = _sc_segment_sum(x, src, dst, w)
    return _tc_combine(partials, x, W_rel, b_rel, W_root)
